# baseline (speedup 1.0000x reference)
import jax
import jax.numpy as jnp
from jax.experimental import pallas as pl
from jax.experimental.pallas import tpu as pltpu

_EPS = 0.01
_N = 4096
_BLK = 256


def _toz_body(x_ref, o_ref):
    i = pl.program_id(0)
    row = i * _BLK + jax.lax.broadcasted_iota(jnp.int32, (_BLK, 1, 64, 64), 0)
    fcode = (jax.lax.broadcasted_iota(jnp.int32, (_BLK, 1, 64, 64), 2) * 64
             + jax.lax.broadcasted_iota(jnp.int32, (_BLK, 1, 64, 64), 3))
    diag = jnp.where(row - 1 == fcode, _EPS, 0.0).astype(jnp.float32)
    o_ref[...] = jnp.where(row == 0, x_ref[...], diag)


def kernel(x):
    grid = (_N + 1 + _BLK - 1) // _BLK
    out = pl.pallas_call(
        _toz_body,
        grid=(grid,),
        in_specs=[pl.BlockSpec((1, 1, 64, 64), lambda i: (0, 0, 0, 0))],
        out_specs=pl.BlockSpec((_BLK, 1, 64, 64), lambda i: (i, 0, 0, 0)),
        out_shape=jax.ShapeDtypeStruct((_N + 1, 1, 64, 64), jnp.float32),
    )(x)
    return out


# speedup vs baseline: 1.7935x; 1.7935x over previous
import jax
import jax.numpy as jnp
from jax.experimental import pallas as pl
from jax.experimental.pallas import tpu as pltpu

_EPS = 0.01
_N = 4096
_BLK = 256


def _toz_body(x_ref, o_ref):
    i = pl.program_id(0)
    row = i * _BLK + jax.lax.broadcasted_iota(jnp.int32, (_BLK, _N), 0)
    col = jax.lax.broadcasted_iota(jnp.int32, (_BLK, _N), 1)
    diag = jnp.where(row - 1 == col, _EPS, 0.0).astype(jnp.float32)
    o_ref[...] = jnp.where(row == 0, x_ref[...], diag)


def kernel(x):
    xf = x.reshape(1, _N)
    grid = (_N + 1 + _BLK - 1) // _BLK
    out = pl.pallas_call(
        _toz_body,
        grid=(grid,),
        in_specs=[pl.BlockSpec((1, _N), lambda i: (0, 0))],
        out_specs=pl.BlockSpec((_BLK, _N), lambda i: (i, 0)),
        out_shape=jax.ShapeDtypeStruct((_N + 1, _N), jnp.float32),
    )(xf)
    return out.reshape(_N + 1, 1, 64, 64)
